# baseline (speedup 1.0000x reference)
import jax
import jax.numpy as jnp
import numpy as np
from jax.experimental import pallas as pl

B, N, FEAT, KNN, NPROP = 2, 1024, 256, 32, 64
NS = 27
M = NPROP * NS
QC = 864
PAIRS = NS * KNN


def _proto():
    s = []
    for i in range(3):
        for j in range(3):
            for k in range(3):
                s.append(((i + 0.5) / 3.0, (j + 0.5) / 3.0, (k + 0.5) / 3.0))
    return np.asarray(s, dtype=np.float32) - 0.5


_PROTO = _proto()


def _dot(a, b):
    return jnp.dot(a, b, precision=jax.lax.Precision.DEFAULT,
                   preferred_element_type=jnp.float32)


def _dotx(a, b):
    return jnp.dot(a, b, precision=jax.lax.Precision.HIGHEST,
                   preferred_element_type=jnp.float32)


def _scale(g):
    return g / jnp.sqrt(jnp.float32(1.0 + 1e-5))


def _stage_a_body(geo_ref, mask_ref, xyzt_ref,
                  fmW1, fmg1, fmb1, fmW2, fmg2, fmb2, fmW3, fmb3,
                  ccW1, ccg1, ccb1, ccW2, ccg2, ccb2, ccW3, ccb3,
                  pmW1, pmg1, pmb1, pmW2, pmb2,
                  smW1, smg1,
                  mask_out, obj_out, cpt_out, gt_out):
    feat = geo_ref[0] + mask_ref[0]
    xyzt = xyzt_ref[0]

    h = jnp.maximum(_dot(fmW1[...], feat) * _scale(fmg1[...])
                    + fmb1[...], 0.0)
    h = jnp.maximum(_dot(fmW2[...], h) * _scale(fmg2[...])
                    + fmb2[...], 0.0)
    mask_pred = _dot(fmW3[...], h) + fmb3[...]
    mask_out[0] = mask_pred

    h = jnp.maximum((_dot(ccW1[...][:, :FEAT], feat)
                     + _dot(ccW1[...][:, FEAT:FEAT + 3], xyzt))
                    * _scale(ccg1[...]) + ccb1[...], 0.0)
    h = jnp.maximum(_dot(ccW2[...], h) * _scale(ccg2[...])
                    + ccb2[...], 0.0)
    offset = _dot(ccW3[...], h) + ccb3[...]

    feat2 = feat + offset[:FEAT]
    cpt_out[0] = offset[FEAT:FEAT + 3] + xyzt
    obj_out[0] = offset[FEAT + 3:FEAT + 4]

    sig = jax.nn.sigmoid(mask_pred)
    h = jnp.maximum((_dot(pmW1[...][:, 0:1], sig)
                     + _dot(pmW1[...][:, 1:FEAT + 1], feat2))
                    * _scale(pmg1[...]) + pmb1[...], 0.0)
    pf = _dot(pmW2[...], h) + pmb2[...]

    g = (_dot(smW1[...][:, 0:3], xyzt)
         + _dot(smW1[...][:, 6:6 + FEAT], pf))
    gt_out[0] = jnp.transpose(g)


def _fps_body(cpt_ref, obj_ref, lwh_ref, proto_ref,
              prop_out, objs_out, samp_out):
    iota2 = (jax.lax.broadcasted_iota(jnp.int32, (8, 128), 0) * 128
             + jax.lax.broadcasted_iota(jnp.int32, (8, 128), 1))
    sub64 = jax.lax.broadcasted_iota(jnp.int32, (NPROP, 1), 0)
    pts_all = []
    st = []
    z = jnp.zeros((NPROP, 1), jnp.float32)
    for b in range(B):
        ct = cpt_ref[b]
        pts_all.append((ct[0:1, :].reshape(8, 128),
                        ct[1:2, :].reshape(8, 128),
                        ct[2:3, :].reshape(8, 128),
                        obj_ref[b].reshape(8, 128)))
        st.append((jnp.full((8, 128), 1e10, jnp.float32),
                   jnp.asarray(0, jnp.int32), z, z, z, z))

    for j in range(NPROP):
        sel = sub64 == j
        for b in range(B):
            dists, idx, px, py, pz, pobj = st[b]
            cx, cy, cz, objr = pts_all[b]
            msk = iota2 == idx
            lx = jnp.sum(jnp.where(msk, cx, 0.0))
            ly = jnp.sum(jnp.where(msk, cy, 0.0))
            lz = jnp.sum(jnp.where(msk, cz, 0.0))
            lo = jnp.sum(jnp.where(msk, objr, 0.0))
            px = jnp.where(sel, lx, px)
            py = jnp.where(sel, ly, py)
            pz = jnp.where(sel, lz, pz)
            pobj = jnp.where(sel, lo, pobj)
            if j < NPROP - 1:
                d2 = (cx - lx) ** 2 + (cy - ly) ** 2 + (cz - lz) ** 2
                dists = jnp.minimum(dists, d2)
                mx = jnp.max(dists)
                idx = jnp.min(jnp.where(dists == mx, iota2, N + 1))
            st[b] = (dists, idx, px, py, pz, pobj)

    for b in range(B):
        _, _, px, py, pz, pobj = st[b]
        objs_out[b] = jax.nn.sigmoid(pobj)
        prop = jnp.concatenate([px, py, pz], axis=1)
        prop_out[b] = prop
        pts = proto_ref[...] * lwh_ref[b]
        samp_out[b] = prop[:, None, :] + pts[None, :, :]


def _knn_body(s_ref, xt_ref, idx_out):
    s = s_ref[0]
    xt = xt_ref[0]
    dx = s[:, 0:1] - xt[0:1, :]
    dy = s[:, 1:2] - xt[1:2, :]
    dz = s[:, 2:3] - xt[2:3, :]
    d2 = dx * dx + dy * dy + dz * dz
    iotaf = jax.lax.broadcasted_iota(jnp.int32, (QC, N), 1).astype(jnp.float32)
    big = jnp.float32(N + 1)
    for j in range(KNN):
        m = jnp.min(d2, axis=1, keepdims=True)
        amf = jnp.min(jnp.where(d2 == m, iotaf, big), axis=1, keepdims=True)
        idx_out[0, :, j] = amf[:, 0].astype(jnp.int32)
        d2 = jnp.where(iotaf == amf, jnp.float32(1e30), d2)


def _edge_body(gt_ref, idx_ref, s_ref, w1t_ref, g1r, b1r, w2t_ref, g2r, b2r,
               nf_out):
    gt = gt_ref[0]
    idxcol = idx_ref[0, 0]
    sx = s_ref[0, 0]
    w1t = w1t_ref[...]
    cqt = _dotx(sx, w1t[3:6] - w1t[0:3])

    oh = (jax.lax.broadcasted_iota(jnp.int32, (PAIRS, N), 1)
          == idxcol).astype(jnp.float32)
    ghi = gt.astype(jnp.bfloat16).astype(jnp.float32)
    gath = _dot(oh, ghi) + _dot(oh, gt - ghi)
    h1 = jnp.maximum((gath.reshape(NS, KNN, FEAT) + cqt[:, None, :])
                     * _scale(g1r[...]) + b1r[...], 0.0)
    y = (_dot(h1.reshape(PAIRS, FEAT), w2t_ref[...])
         * _scale(g2r[...]) + b2r[...])
    y = jnp.maximum(y, 0.0).reshape(NS, KNN, FEAT)
    nf_out[0, 0] = jnp.max(y, axis=1)


def _head_body(nf_ref, objs_ref, prop_ref, ag3t_ref, agg, agb, ceWr, cebr,
               fpW1t, fpg1, fpb1, fpW2t, fpg2, fpb2, fpW3t, fpb3,
               boxes_out):
    nf = nf_ref[0]
    acc = _dot(nf[:, 0, :], ag3t_ref[0])
    for s in range(1, NS):
        acc = acc + _dot(nf[:, s, :], ag3t_ref[s])
    pfeat = jnp.maximum(acc * _scale(agg[...]) + agb[...], 0.0)
    ce = _dot(objs_ref[0], ceWr[...]) + cebr[...]
    x = pfeat + ce
    h = jnp.maximum(_dot(x, fpW1t[...]) * _scale(fpg1[...])
                    + fpb1[...], 0.0)
    h = jnp.maximum(_dot(h, fpW2t[...]) * _scale(fpg2[...])
                    + fpb2[...], 0.0)
    po = _dot(h, fpW3t[...]) + fpb3[...]
    prop = prop_ref[0]
    boxes_out[0] = jnp.concatenate([po[:, 0:3] + prop, po[:, 3:5]], axis=1)


def _full(shape):
    nd = len(shape)
    return pl.BlockSpec(shape, lambda *_: (0,) * nd)


def _batched(shape):
    nd = len(shape)

    def imap(b, *_):
        return (b,) + (0,) * nd

    return pl.BlockSpec((1,) + shape, imap)


def kernel(xyz, geo_feat, mask_feat, lwh,
           fm_W1, fm_g1, fm_b1, fm_W2, fm_g2, fm_b2, fm_W3, fm_bias3,
           cc_W1, cc_g1, cc_b1, cc_W2, cc_g2, cc_b2, cc_W3, cc_bias3,
           ce_W, ce_b,
           pm_W1, pm_g1, pm_b1, pm_W2, pm_bias2,
           sm_W1, sm_g1, sm_b1, sm_W2, sm_g2, sm_b2,
           ag_W, ag_g, ag_b,
           fp_W1, fp_g1, fp_b1, fp_W2, fp_g2, fp_b2, fp_W3, fp_bias3):
    f32 = jnp.float32
    col = lambda v: v.reshape(-1, 1)
    row = lambda v: v.reshape(1, -1)
    xyzt = jnp.transpose(xyz, (0, 2, 1))

    mask_pred2, obj2, cpt, Gt = pl.pallas_call(
        _stage_a_body,
        grid=(B,),
        in_specs=[_batched((FEAT, N)), _batched((FEAT, N)), _batched((3, N))]
        + [_full(s) for s in [
            (FEAT, FEAT), (FEAT, 1), (FEAT, 1),
            (FEAT, FEAT), (FEAT, 1), (FEAT, 1), (1, FEAT), (1, 1),
            (FEAT, FEAT + 3), (FEAT, 1), (FEAT, 1),
            (FEAT, FEAT), (FEAT, 1), (FEAT, 1),
            (FEAT + 4, FEAT), (FEAT + 4, 1),
            (FEAT, FEAT + 1), (FEAT, 1), (FEAT, 1),
            (FEAT, FEAT), (FEAT, 1),
            (FEAT, FEAT + 6), (FEAT, 1)]],
        out_specs=[_batched((1, N)), _batched((1, N)), _batched((3, N)),
                   _batched((N, FEAT))],
        out_shape=[jax.ShapeDtypeStruct((B, 1, N), f32),
                   jax.ShapeDtypeStruct((B, 1, N), f32),
                   jax.ShapeDtypeStruct((B, 3, N), f32),
                   jax.ShapeDtypeStruct((B, N, FEAT), f32)],
    )(geo_feat, mask_feat, xyzt,
      fm_W1, col(fm_g1), col(fm_b1), fm_W2, col(fm_g2), col(fm_b2),
      fm_W3, col(fm_bias3),
      cc_W1, col(cc_g1), col(cc_b1), cc_W2, col(cc_g2), col(cc_b2),
      cc_W3, col(cc_bias3),
      pm_W1, col(pm_g1), col(pm_b1), pm_W2, col(pm_bias2),
      sm_W1, col(sm_g1))

    propt, objs, samplet = pl.pallas_call(
        _fps_body,
        out_shape=[jax.ShapeDtypeStruct((B, NPROP, 3), f32),
                   jax.ShapeDtypeStruct((B, NPROP, 1), f32),
                   jax.ShapeDtypeStruct((B, NPROP, NS, 3), f32)],
    )(cpt, obj2, lwh.reshape(B, 1, 3), jnp.asarray(_PROTO))

    sample = samplet.reshape(B, M, 3)

    knn_idx = pl.pallas_call(
        _knn_body,
        grid=(B, M // QC),
        in_specs=[pl.BlockSpec((1, QC, 3), lambda b, q: (b, q, 0)),
                  pl.BlockSpec((1, 3, N), lambda b, q: (b, 0, 0))],
        out_specs=pl.BlockSpec((1, QC, KNN), lambda b, q: (b, q, 0)),
        out_shape=jax.ShapeDtypeStruct((B, M, KNN), jnp.int32),
    )(sample, xyzt)

    knn_col = knn_idx.reshape(B, NPROP, PAIRS, 1)

    nf4 = pl.pallas_call(
        _edge_body,
        grid=(B, NPROP),
        in_specs=[pl.BlockSpec((1, N, FEAT), lambda b, p: (b, 0, 0)),
                  pl.BlockSpec((1, 1, PAIRS, 1), lambda b, p: (b, p, 0, 0)),
                  pl.BlockSpec((1, 1, NS, 3), lambda b, p: (b, p, 0, 0)),
                  _full((FEAT + 6, FEAT)), _full((1, FEAT)), _full((1, FEAT)),
                  _full((FEAT, FEAT)), _full((1, FEAT)), _full((1, FEAT))],
        out_specs=pl.BlockSpec((1, 1, NS, FEAT), lambda b, p: (b, p, 0, 0)),
        out_shape=jax.ShapeDtypeStruct((B, NPROP, NS, FEAT), f32),
    )(Gt, knn_col, samplet,
      jnp.transpose(sm_W1), row(sm_g1), row(sm_b1),
      jnp.transpose(sm_W2), row(sm_g2), row(sm_b2))

    ag3t = jnp.transpose(ag_W.reshape(FEAT, FEAT, NS), (2, 1, 0))
    boxes = pl.pallas_call(
        _head_body,
        grid=(B,),
        in_specs=[_batched((NPROP, NS, FEAT)), _batched((NPROP, 1)),
                  _batched((NPROP, 3)),
                  _full((NS, FEAT, FEAT)), _full((1, FEAT)), _full((1, FEAT)),
                  _full((1, FEAT)), _full((1, FEAT)),
                  _full((FEAT, FEAT)), _full((1, FEAT)), _full((1, FEAT)),
                  _full((FEAT, FEAT)), _full((1, FEAT)), _full((1, FEAT)),
                  _full((FEAT, 5)), _full((1, 5))],
        out_specs=_batched((NPROP, 5)),
        out_shape=jax.ShapeDtypeStruct((B, NPROP, 5), f32),
    )(nf4, objs, propt, ag3t, row(ag_g), row(ag_b),
      jnp.transpose(ce_W), row(ce_b),
      jnp.transpose(fp_W1), row(fp_g1), row(fp_b1),
      jnp.transpose(fp_W2), row(fp_g2), row(fp_b2),
      jnp.transpose(fp_W3), row(fp_bias3))

    mask_pred = mask_pred2[:, 0, :]
    objectness_pred = obj2[:, 0, :]
    center_pred = jnp.transpose(cpt, (0, 2, 1))
    return mask_pred, objectness_pred, center_pred, boxes, propt


# speedup vs baseline: 1.1315x; 1.1315x over previous
import jax
import jax.numpy as jnp
import numpy as np
from jax.experimental import pallas as pl

B, N, FEAT, KNN, NPROP = 2, 1024, 256, 32, 64
NS = 27
M = NPROP * NS
QC = 864
PAIRS = NS * KNN


def _proto():
    s = []
    for i in range(3):
        for j in range(3):
            for k in range(3):
                s.append(((i + 0.5) / 3.0, (j + 0.5) / 3.0, (k + 0.5) / 3.0))
    return np.asarray(s, dtype=np.float32) - 0.5


_PROTO = _proto()


def _dot(a, b):
    return jnp.dot(a, b, precision=jax.lax.Precision.DEFAULT,
                   preferred_element_type=jnp.float32)


def _dotx(a, b):
    return jnp.dot(a, b, precision=jax.lax.Precision.HIGHEST,
                   preferred_element_type=jnp.float32)


def _scale(g):
    return g / jnp.sqrt(jnp.float32(1.0 + 1e-5))


def _stage_a_body(geo_ref, mask_ref, xyzt_ref,
                  fmW1, fmg1, fmb1, fmW2, fmg2, fmb2, fmW3, fmb3,
                  ccW1, ccg1, ccb1, ccW2, ccg2, ccb2, ccW3, ccb3,
                  pmW1, pmg1, pmb1, pmW2, pmb2,
                  smW1, smg1,
                  mask_out, obj_out, cpt_out, gt_out):
    feat = geo_ref[0] + mask_ref[0]
    xyzt = xyzt_ref[0]

    h = jnp.maximum(_dot(fmW1[...], feat) * _scale(fmg1[...])
                    + fmb1[...], 0.0)
    h = jnp.maximum(_dot(fmW2[...], h) * _scale(fmg2[...])
                    + fmb2[...], 0.0)
    mask_pred = _dot(fmW3[...], h) + fmb3[...]
    mask_out[0] = mask_pred

    h = jnp.maximum((_dot(ccW1[...][:, :FEAT], feat)
                     + _dot(ccW1[...][:, FEAT:FEAT + 3], xyzt))
                    * _scale(ccg1[...]) + ccb1[...], 0.0)
    h = jnp.maximum(_dot(ccW2[...], h) * _scale(ccg2[...])
                    + ccb2[...], 0.0)
    offset = _dot(ccW3[...], h) + ccb3[...]

    feat2 = feat + offset[:FEAT]
    cpt_out[0] = offset[FEAT:FEAT + 3] + xyzt
    obj_out[0] = offset[FEAT + 3:FEAT + 4]

    sig = jax.nn.sigmoid(mask_pred)
    h = jnp.maximum((_dot(pmW1[...][:, 0:1], sig)
                     + _dot(pmW1[...][:, 1:FEAT + 1], feat2))
                    * _scale(pmg1[...]) + pmb1[...], 0.0)
    pf = _dot(pmW2[...], h) + pmb2[...]

    g = (_dot(smW1[...][:, 0:3], xyzt)
         + _dot(smW1[...][:, 6:6 + FEAT], pf))
    gt_out[0] = jnp.transpose(g)


def _fps_body(cpt_ref, obj_ref, lwh_ref, proto_ref,
              prop_out, objs_out, samp_out):
    iota2 = (jax.lax.broadcasted_iota(jnp.int32, (8, 128), 0) * 128
             + jax.lax.broadcasted_iota(jnp.int32, (8, 128), 1))
    sub64 = jax.lax.broadcasted_iota(jnp.int32, (NPROP, 1), 0)
    pts_all = []
    st = []
    z = jnp.zeros((NPROP, 1), jnp.float32)
    for b in range(B):
        ct = cpt_ref[b]
        pts_all.append((ct[0:1, :].reshape(8, 128),
                        ct[1:2, :].reshape(8, 128),
                        ct[2:3, :].reshape(8, 128),
                        obj_ref[b].reshape(8, 128)))
        st.append((jnp.full((8, 128), 1e10, jnp.float32),
                   jnp.asarray(0, jnp.int32), z, z, z, z))

    for j in range(NPROP):
        sel = sub64 == j
        for b in range(B):
            dists, idx, px, py, pz, pobj = st[b]
            cx, cy, cz, objr = pts_all[b]
            msk = iota2 == idx
            lx = jnp.sum(jnp.where(msk, cx, 0.0))
            ly = jnp.sum(jnp.where(msk, cy, 0.0))
            lz = jnp.sum(jnp.where(msk, cz, 0.0))
            lo = jnp.sum(jnp.where(msk, objr, 0.0))
            px = jnp.where(sel, lx, px)
            py = jnp.where(sel, ly, py)
            pz = jnp.where(sel, lz, pz)
            pobj = jnp.where(sel, lo, pobj)
            if j < NPROP - 1:
                d2 = (cx - lx) ** 2 + (cy - ly) ** 2 + (cz - lz) ** 2
                dists = jnp.minimum(dists, d2)
                mx = jnp.max(dists)
                idx = jnp.min(jnp.where(dists == mx, iota2, N + 1))
            st[b] = (dists, idx, px, py, pz, pobj)

    for b in range(B):
        _, _, px, py, pz, pobj = st[b]
        objs_out[b] = jax.nn.sigmoid(pobj)
        prop = jnp.concatenate([px, py, pz], axis=1)
        prop_out[b] = prop
        pts = proto_ref[...] * lwh_ref[b]
        samp_out[b] = prop[:, None, :] + pts[None, :, :]


def _knn_body(s_ref, xt_ref, idx_out):
    s = s_ref[0]
    xt = xt_ref[0]
    dx = s[:, 0:1] - xt[0:1, :]
    dy = s[:, 1:2] - xt[1:2, :]
    dz = s[:, 2:3] - xt[2:3, :]
    d2 = dx * dx + dy * dy + dz * dz
    iotaf = jax.lax.broadcasted_iota(jnp.int32, (QC, N), 1).astype(jnp.float32)
    big = jnp.float32(N + 1)
    for j in range(KNN):
        m = jnp.min(d2, axis=1, keepdims=True)
        amf = jnp.min(jnp.where(d2 == m, iotaf, big), axis=1, keepdims=True)
        idx_out[0, :, j] = amf[:, 0].astype(jnp.int32)
        d2 = jnp.where(iotaf == amf, jnp.float32(1e30), d2)


def _edge_body(gt_ref, idx_ref, s_ref, w1t_ref, g1r, b1r, w2t_ref, g2r, b2r,
               nf_out):
    gt = gt_ref[0]
    idxcol = idx_ref[0, 0]
    sx = s_ref[0, 0]
    w1t = w1t_ref[...]
    cqt = _dotx(sx, w1t[3:6] - w1t[0:3])

    oh = (jax.lax.broadcasted_iota(jnp.int32, (PAIRS, N), 1)
          == idxcol).astype(jnp.float32)
    gath = _dot(oh, gt)
    h1 = jnp.maximum((gath.reshape(NS, KNN, FEAT) + cqt[:, None, :])
                     * _scale(g1r[...]) + b1r[...], 0.0)
    y = (_dot(h1.reshape(PAIRS, FEAT), w2t_ref[...])
         * _scale(g2r[...]) + b2r[...])
    y = jnp.maximum(y, 0.0).reshape(NS, KNN, FEAT)
    nf_out[0, 0] = jnp.max(y, axis=1)


def _head_body(nf_ref, objs_ref, prop_ref, ag3t_ref, agg, agb, ceWr, cebr,
               fpW1t, fpg1, fpb1, fpW2t, fpg2, fpb2, fpW3t, fpb3,
               boxes_out):
    nf = nf_ref[0]
    acc = _dot(nf[:, 0, :], ag3t_ref[0])
    for s in range(1, NS):
        acc = acc + _dot(nf[:, s, :], ag3t_ref[s])
    pfeat = jnp.maximum(acc * _scale(agg[...]) + agb[...], 0.0)
    ce = _dot(objs_ref[0], ceWr[...]) + cebr[...]
    x = pfeat + ce
    h = jnp.maximum(_dot(x, fpW1t[...]) * _scale(fpg1[...])
                    + fpb1[...], 0.0)
    h = jnp.maximum(_dot(h, fpW2t[...]) * _scale(fpg2[...])
                    + fpb2[...], 0.0)
    po = _dot(h, fpW3t[...]) + fpb3[...]
    prop = prop_ref[0]
    boxes_out[0] = jnp.concatenate([po[:, 0:3] + prop, po[:, 3:5]], axis=1)


def _full(shape):
    nd = len(shape)
    return pl.BlockSpec(shape, lambda *_: (0,) * nd)


def _batched(shape):
    nd = len(shape)

    def imap(b, *_):
        return (b,) + (0,) * nd

    return pl.BlockSpec((1,) + shape, imap)


def kernel(xyz, geo_feat, mask_feat, lwh,
           fm_W1, fm_g1, fm_b1, fm_W2, fm_g2, fm_b2, fm_W3, fm_bias3,
           cc_W1, cc_g1, cc_b1, cc_W2, cc_g2, cc_b2, cc_W3, cc_bias3,
           ce_W, ce_b,
           pm_W1, pm_g1, pm_b1, pm_W2, pm_bias2,
           sm_W1, sm_g1, sm_b1, sm_W2, sm_g2, sm_b2,
           ag_W, ag_g, ag_b,
           fp_W1, fp_g1, fp_b1, fp_W2, fp_g2, fp_b2, fp_W3, fp_bias3):
    f32 = jnp.float32
    col = lambda v: v.reshape(-1, 1)
    row = lambda v: v.reshape(1, -1)
    xyzt = jnp.transpose(xyz, (0, 2, 1))

    mask_pred2, obj2, cpt, Gt = pl.pallas_call(
        _stage_a_body,
        grid=(B,),
        in_specs=[_batched((FEAT, N)), _batched((FEAT, N)), _batched((3, N))]
        + [_full(s) for s in [
            (FEAT, FEAT), (FEAT, 1), (FEAT, 1),
            (FEAT, FEAT), (FEAT, 1), (FEAT, 1), (1, FEAT), (1, 1),
            (FEAT, FEAT + 3), (FEAT, 1), (FEAT, 1),
            (FEAT, FEAT), (FEAT, 1), (FEAT, 1),
            (FEAT + 4, FEAT), (FEAT + 4, 1),
            (FEAT, FEAT + 1), (FEAT, 1), (FEAT, 1),
            (FEAT, FEAT), (FEAT, 1),
            (FEAT, FEAT + 6), (FEAT, 1)]],
        out_specs=[_batched((1, N)), _batched((1, N)), _batched((3, N)),
                   _batched((N, FEAT))],
        out_shape=[jax.ShapeDtypeStruct((B, 1, N), f32),
                   jax.ShapeDtypeStruct((B, 1, N), f32),
                   jax.ShapeDtypeStruct((B, 3, N), f32),
                   jax.ShapeDtypeStruct((B, N, FEAT), f32)],
    )(geo_feat, mask_feat, xyzt,
      fm_W1, col(fm_g1), col(fm_b1), fm_W2, col(fm_g2), col(fm_b2),
      fm_W3, col(fm_bias3),
      cc_W1, col(cc_g1), col(cc_b1), cc_W2, col(cc_g2), col(cc_b2),
      cc_W3, col(cc_bias3),
      pm_W1, col(pm_g1), col(pm_b1), pm_W2, col(pm_bias2),
      sm_W1, col(sm_g1))

    propt, objs, samplet = pl.pallas_call(
        _fps_body,
        out_shape=[jax.ShapeDtypeStruct((B, NPROP, 3), f32),
                   jax.ShapeDtypeStruct((B, NPROP, 1), f32),
                   jax.ShapeDtypeStruct((B, NPROP, NS, 3), f32)],
    )(cpt, obj2, lwh.reshape(B, 1, 3), jnp.asarray(_PROTO))

    sample = samplet.reshape(B, M, 3)

    knn_idx = pl.pallas_call(
        _knn_body,
        grid=(B, M // QC),
        in_specs=[pl.BlockSpec((1, QC, 3), lambda b, q: (b, q, 0)),
                  pl.BlockSpec((1, 3, N), lambda b, q: (b, 0, 0))],
        out_specs=pl.BlockSpec((1, QC, KNN), lambda b, q: (b, q, 0)),
        out_shape=jax.ShapeDtypeStruct((B, M, KNN), jnp.int32),
    )(sample, xyzt)

    knn_col = knn_idx.reshape(B, NPROP, PAIRS, 1)

    nf4 = pl.pallas_call(
        _edge_body,
        grid=(B, NPROP),
        in_specs=[pl.BlockSpec((1, N, FEAT), lambda b, p: (b, 0, 0)),
                  pl.BlockSpec((1, 1, PAIRS, 1), lambda b, p: (b, p, 0, 0)),
                  pl.BlockSpec((1, 1, NS, 3), lambda b, p: (b, p, 0, 0)),
                  _full((FEAT + 6, FEAT)), _full((1, FEAT)), _full((1, FEAT)),
                  _full((FEAT, FEAT)), _full((1, FEAT)), _full((1, FEAT))],
        out_specs=pl.BlockSpec((1, 1, NS, FEAT), lambda b, p: (b, p, 0, 0)),
        out_shape=jax.ShapeDtypeStruct((B, NPROP, NS, FEAT), f32),
    )(Gt, knn_col, samplet,
      jnp.transpose(sm_W1), row(sm_g1), row(sm_b1),
      jnp.transpose(sm_W2), row(sm_g2), row(sm_b2))

    ag3t = jnp.transpose(ag_W.reshape(FEAT, FEAT, NS), (2, 1, 0))
    boxes = pl.pallas_call(
        _head_body,
        grid=(B,),
        in_specs=[_batched((NPROP, NS, FEAT)), _batched((NPROP, 1)),
                  _batched((NPROP, 3)),
                  _full((NS, FEAT, FEAT)), _full((1, FEAT)), _full((1, FEAT)),
                  _full((1, FEAT)), _full((1, FEAT)),
                  _full((FEAT, FEAT)), _full((1, FEAT)), _full((1, FEAT)),
                  _full((FEAT, FEAT)), _full((1, FEAT)), _full((1, FEAT)),
                  _full((FEAT, 5)), _full((1, 5))],
        out_specs=_batched((NPROP, 5)),
        out_shape=jax.ShapeDtypeStruct((B, NPROP, 5), f32),
    )(nf4, objs, propt, ag3t, row(ag_g), row(ag_b),
      jnp.transpose(ce_W), row(ce_b),
      jnp.transpose(fp_W1), row(fp_g1), row(fp_b1),
      jnp.transpose(fp_W2), row(fp_g2), row(fp_b2),
      jnp.transpose(fp_W3), row(fp_bias3))

    mask_pred = mask_pred2[:, 0, :]
    objectness_pred = obj2[:, 0, :]
    center_pred = jnp.transpose(cpt, (0, 2, 1))
    return mask_pred, objectness_pred, center_pred, boxes, propt


# speedup vs baseline: 1.2854x; 1.1360x over previous
import jax
import jax.numpy as jnp
import numpy as np
from jax.experimental import pallas as pl

B, N, FEAT, KNN, NPROP = 2, 1024, 256, 32, 64
NS = 27
M = NPROP * NS
PC = 32
NB = NPROP // PC
PAIRS = NS * KNN


def _proto():
    s = []
    for i in range(3):
        for j in range(3):
            for k in range(3):
                s.append(((i + 0.5) / 3.0, (j + 0.5) / 3.0, (k + 0.5) / 3.0))
    return np.asarray(s, dtype=np.float32) - 0.5


_PROTO = _proto()


def _dot(a, b):
    return jnp.dot(a, b, precision=jax.lax.Precision.DEFAULT,
                   preferred_element_type=jnp.float32)


def _dotx(a, b):
    return jnp.dot(a, b, precision=jax.lax.Precision.HIGHEST,
                   preferred_element_type=jnp.float32)


def _scale(g):
    return g / jnp.sqrt(jnp.float32(1.0 + 1e-5))


def _abc_body(geo_ref, mask_ref, xyzt_ref, lwh_ref, proto_ref,
              fmW1, fmg1, fmb1, fmW2, fmg2, fmb2, fmW3, fmb3,
              ccW1, ccg1, ccb1, ccW2, ccg2, ccb2, ccW3, ccb3,
              pmW1, pmg1, pmb1, pmW2, pmb2,
              smW1, smg1,
              mask_out, obj_out, cpt_out, gt_out,
              prop_out, objs_out, samp_out, knn_out):
    cpt_all, obj_all = [], []
    for b in range(B):
        feat = geo_ref[b] + mask_ref[b]
        xyzt = xyzt_ref[b]

        h = jnp.maximum(_dot(fmW1[...], feat) * _scale(fmg1[...])
                        + fmb1[...], 0.0)
        h = jnp.maximum(_dot(fmW2[...], h) * _scale(fmg2[...])
                        + fmb2[...], 0.0)
        mask_pred = _dot(fmW3[...], h) + fmb3[...]
        mask_out[b] = mask_pred

        h = jnp.maximum((_dot(ccW1[...][:, :FEAT], feat)
                         + _dot(ccW1[...][:, FEAT:FEAT + 3], xyzt))
                        * _scale(ccg1[...]) + ccb1[...], 0.0)
        h = jnp.maximum(_dot(ccW2[...], h) * _scale(ccg2[...])
                        + ccb2[...], 0.0)
        offset = _dot(ccW3[...], h) + ccb3[...]

        feat2 = feat + offset[:FEAT]
        cpt = offset[FEAT:FEAT + 3] + xyzt
        cpt_out[b] = cpt
        cpt_all.append(cpt)
        obj = offset[FEAT + 3:FEAT + 4]
        obj_out[b] = obj
        obj_all.append(obj)

        sig = jax.nn.sigmoid(mask_pred)
        h = jnp.maximum((_dot(pmW1[...][:, 0:1], sig)
                         + _dot(pmW1[...][:, 1:FEAT + 1], feat2))
                        * _scale(pmg1[...]) + pmb1[...], 0.0)
        pf = _dot(pmW2[...], h) + pmb2[...]

        g = (_dot(smW1[...][:, 0:3], xyzt)
             + _dot(smW1[...][:, 6:6 + FEAT], pf))
        gt_out[b] = jnp.transpose(g)

    iota2 = (jax.lax.broadcasted_iota(jnp.int32, (8, 128), 0) * 128
             + jax.lax.broadcasted_iota(jnp.int32, (8, 128), 1))
    sub64 = jax.lax.broadcasted_iota(jnp.int32, (NPROP, 1), 0)
    z = jnp.zeros((NPROP, 1), jnp.float32)
    pts_all, st = [], []
    for b in range(B):
        ct = cpt_all[b]
        pts_all.append((ct[0:1, :].reshape(8, 128),
                        ct[1:2, :].reshape(8, 128),
                        ct[2:3, :].reshape(8, 128),
                        obj_all[b].reshape(8, 128)))
        st.append((jnp.full((8, 128), 1e10, jnp.float32),
                   jnp.asarray(0, jnp.int32), z, z, z, z))

    for j in range(NPROP):
        sel = sub64 == j
        for b in range(B):
            dists, idx, px, py, pz, pobj = st[b]
            cx, cy, cz, objr = pts_all[b]
            msk = iota2 == idx
            lx = jnp.sum(jnp.where(msk, cx, 0.0))
            ly = jnp.sum(jnp.where(msk, cy, 0.0))
            lz = jnp.sum(jnp.where(msk, cz, 0.0))
            lo = jnp.sum(jnp.where(msk, objr, 0.0))
            px = jnp.where(sel, lx, px)
            py = jnp.where(sel, ly, py)
            pz = jnp.where(sel, lz, pz)
            pobj = jnp.where(sel, lo, pobj)
            if j < NPROP - 1:
                d2 = (cx - lx) ** 2 + (cy - ly) ** 2 + (cz - lz) ** 2
                dists = jnp.minimum(dists, d2)
                mx = jnp.max(dists)
                idx = jnp.min(jnp.where(dists == mx, iota2, N + 1))
            st[b] = (dists, idx, px, py, pz, pobj)

    samp_all = []
    for b in range(B):
        _, _, px, py, pz, pobj = st[b]
        objs_out[b] = jax.nn.sigmoid(pobj)
        prop = jnp.concatenate([px, py, pz], axis=1)
        prop_out[b] = prop
        pts = proto_ref[...] * lwh_ref[b]
        samp = prop[:, None, :] + pts[None, :, :]
        samp_out[b] = samp
        samp_all.append(samp)

    big = jnp.float32(N + 1)
    iotaf = (jax.lax.broadcasted_iota(jnp.int32, (PC, NS, N), 2)
             .astype(jnp.float32))
    for b in range(B):
        xt = xyzt_ref[b]
        xr = [xt[c:c + 1, :][None, :, :] for c in range(3)]
        for q in range(NB):
            s3 = samp_all[b][q * PC:(q + 1) * PC]
            dx = s3[:, :, 0:1] - xr[0]
            dy = s3[:, :, 1:2] - xr[1]
            dz = s3[:, :, 2:3] - xr[2]
            d2 = dx * dx + dy * dy + dz * dz
            for j in range(KNN):
                m = jnp.min(d2, axis=2, keepdims=True)
                amf = jnp.min(jnp.where(d2 == m, iotaf, big),
                              axis=2, keepdims=True)
                knn_out[b, q * PC:(q + 1) * PC, :, j] = (
                    amf[:, :, 0].astype(jnp.int32))
                d2 = jnp.where(iotaf == amf, jnp.float32(1e30), d2)


def _edge_body(gt_ref, idx_ref, s_ref, w1t_ref, g1r, b1r, w2t_ref, g2r, b2r,
               nf_out):
    gt = gt_ref[0]
    idxcol = idx_ref[0, 0]
    sx = s_ref[0, 0]
    w1t = w1t_ref[...]
    cqt = _dotx(sx, w1t[3:6] - w1t[0:3])

    oh = (jax.lax.broadcasted_iota(jnp.int32, (PAIRS, N), 1)
          == idxcol).astype(jnp.float32)
    gath = _dot(oh, gt)
    h1 = jnp.maximum((gath.reshape(NS, KNN, FEAT) + cqt[:, None, :])
                     * _scale(g1r[...]) + b1r[...], 0.0)
    y = (_dot(h1.reshape(PAIRS, FEAT), w2t_ref[...])
         * _scale(g2r[...]) + b2r[...])
    y = jnp.maximum(y, 0.0).reshape(NS, KNN, FEAT)
    nf_out[0, 0] = jnp.max(y, axis=1)


def _head_body(nf_ref, objs_ref, prop_ref, ag3t_ref, agg, agb, ceWr, cebr,
               fpW1t, fpg1, fpb1, fpW2t, fpg2, fpb2, fpW3t, fpb3,
               boxes_out):
    nf = nf_ref[0]
    acc = _dot(nf[:, 0, :], ag3t_ref[0])
    for s in range(1, NS):
        acc = acc + _dot(nf[:, s, :], ag3t_ref[s])
    pfeat = jnp.maximum(acc * _scale(agg[...]) + agb[...], 0.0)
    ce = _dot(objs_ref[0], ceWr[...]) + cebr[...]
    x = pfeat + ce
    h = jnp.maximum(_dot(x, fpW1t[...]) * _scale(fpg1[...])
                    + fpb1[...], 0.0)
    h = jnp.maximum(_dot(h, fpW2t[...]) * _scale(fpg2[...])
                    + fpb2[...], 0.0)
    po = _dot(h, fpW3t[...]) + fpb3[...]
    prop = prop_ref[0]
    boxes_out[0] = jnp.concatenate([po[:, 0:3] + prop, po[:, 3:5]], axis=1)


def _full(shape):
    nd = len(shape)
    return pl.BlockSpec(shape, lambda *_: (0,) * nd)


def _batched(shape):
    nd = len(shape)

    def imap(b, *_):
        return (b,) + (0,) * nd

    return pl.BlockSpec((1,) + shape, imap)


def kernel(xyz, geo_feat, mask_feat, lwh,
           fm_W1, fm_g1, fm_b1, fm_W2, fm_g2, fm_b2, fm_W3, fm_bias3,
           cc_W1, cc_g1, cc_b1, cc_W2, cc_g2, cc_b2, cc_W3, cc_bias3,
           ce_W, ce_b,
           pm_W1, pm_g1, pm_b1, pm_W2, pm_bias2,
           sm_W1, sm_g1, sm_b1, sm_W2, sm_g2, sm_b2,
           ag_W, ag_g, ag_b,
           fp_W1, fp_g1, fp_b1, fp_W2, fp_g2, fp_b2, fp_W3, fp_bias3):
    f32 = jnp.float32
    col = lambda v: v.reshape(-1, 1)
    row = lambda v: v.reshape(1, -1)
    xyzt = jnp.transpose(xyz, (0, 2, 1))

    (mask_pred2, obj2, cpt, Gt, propt, objs, samplet, knn_idx) = pl.pallas_call(
        _abc_body,
        out_shape=[jax.ShapeDtypeStruct((B, 1, N), f32),
                   jax.ShapeDtypeStruct((B, 1, N), f32),
                   jax.ShapeDtypeStruct((B, 3, N), f32),
                   jax.ShapeDtypeStruct((B, N, FEAT), f32),
                   jax.ShapeDtypeStruct((B, NPROP, 3), f32),
                   jax.ShapeDtypeStruct((B, NPROP, 1), f32),
                   jax.ShapeDtypeStruct((B, NPROP, NS, 3), f32),
                   jax.ShapeDtypeStruct((B, NPROP, NS, KNN), jnp.int32)],
    )(geo_feat, mask_feat, xyzt, lwh.reshape(B, 1, 3), jnp.asarray(_PROTO),
      fm_W1, col(fm_g1), col(fm_b1), fm_W2, col(fm_g2), col(fm_b2),
      fm_W3, col(fm_bias3),
      cc_W1, col(cc_g1), col(cc_b1), cc_W2, col(cc_g2), col(cc_b2),
      cc_W3, col(cc_bias3),
      pm_W1, col(pm_g1), col(pm_b1), pm_W2, col(pm_bias2),
      sm_W1, col(sm_g1))

    knn_col = knn_idx.reshape(B, NPROP, PAIRS, 1)

    nf4 = pl.pallas_call(
        _edge_body,
        grid=(B, NPROP),
        in_specs=[pl.BlockSpec((1, N, FEAT), lambda b, p: (b, 0, 0)),
                  pl.BlockSpec((1, 1, PAIRS, 1), lambda b, p: (b, p, 0, 0)),
                  pl.BlockSpec((1, 1, NS, 3), lambda b, p: (b, p, 0, 0)),
                  _full((FEAT + 6, FEAT)), _full((1, FEAT)), _full((1, FEAT)),
                  _full((FEAT, FEAT)), _full((1, FEAT)), _full((1, FEAT))],
        out_specs=pl.BlockSpec((1, 1, NS, FEAT), lambda b, p: (b, p, 0, 0)),
        out_shape=jax.ShapeDtypeStruct((B, NPROP, NS, FEAT), f32),
    )(Gt, knn_col, samplet,
      jnp.transpose(sm_W1), row(sm_g1), row(sm_b1),
      jnp.transpose(sm_W2), row(sm_g2), row(sm_b2))

    ag3t = jnp.transpose(ag_W.reshape(FEAT, FEAT, NS), (2, 1, 0))
    boxes = pl.pallas_call(
        _head_body,
        grid=(B,),
        in_specs=[_batched((NPROP, NS, FEAT)), _batched((NPROP, 1)),
                  _batched((NPROP, 3)),
                  _full((NS, FEAT, FEAT)), _full((1, FEAT)), _full((1, FEAT)),
                  _full((1, FEAT)), _full((1, FEAT)),
                  _full((FEAT, FEAT)), _full((1, FEAT)), _full((1, FEAT)),
                  _full((FEAT, FEAT)), _full((1, FEAT)), _full((1, FEAT)),
                  _full((FEAT, 5)), _full((1, 5))],
        out_specs=_batched((NPROP, 5)),
        out_shape=jax.ShapeDtypeStruct((B, NPROP, 5), f32),
    )(nf4, objs, propt, ag3t, row(ag_g), row(ag_b),
      jnp.transpose(ce_W), row(ce_b),
      jnp.transpose(fp_W1), row(fp_g1), row(fp_b1),
      jnp.transpose(fp_W2), row(fp_g2), row(fp_b2),
      jnp.transpose(fp_W3), row(fp_bias3))

    mask_pred = mask_pred2[:, 0, :]
    objectness_pred = obj2[:, 0, :]
    center_pred = jnp.transpose(cpt, (0, 2, 1))
    return mask_pred, objectness_pred, center_pred, boxes, propt
